# Initial kernel scaffold; baseline (speedup 1.0000x reference)
import jax
import jax.numpy as jnp
from jax.experimental import pallas as pl


def kernel(inputs, response_values, response_indices, emb, gates_w, gates_b, layers):
    raise NotImplementedError("write your pallas kernel here")


# speedup vs baseline: 24.7723x; 24.7723x over previous
import functools
import math

import numpy as np
import jax
import jax.numpy as jnp
from jax import lax
from jax.experimental import pallas as pl
from jax.experimental.pallas import tpu as pltpu
from jax.experimental.pallas import tpu_sc as plsc

F32 = jnp.float32

_NC, _NS, _NL = 2, 16, 16
_NW = _NC * _NS


def _posenc(seq, dim):
    pos = np.arange(seq)[:, None].astype(np.float32)
    div = np.exp(np.arange(0, dim, 2).astype(np.float32) * (-math.log(10000.0) / dim))
    pe = np.zeros((seq, dim), np.float32)
    pe[:, 0::2] = np.sin(pos * div)
    pe[:, 1::2] = np.cos(pos * div)
    return jnp.asarray(pe)


def _scalepe_body(x_ref, p_ref, o_ref, *, scale):
    o_ref[...] = x_ref[...] * scale + p_ref[...]


def _scale_pe(x, pe, scale, mb=256):
    M, D = x.shape
    return pl.pallas_call(
        functools.partial(_scalepe_body, scale=scale),
        grid=(M // mb,),
        in_specs=[
            pl.BlockSpec((mb, D), lambda i: (i, 0)),
            pl.BlockSpec((mb, D), lambda i: (i, 0)),
        ],
        out_specs=pl.BlockSpec((mb, D), lambda i: (i, 0)),
        out_shape=jax.ShapeDtypeStruct((M, D), F32),
    )(x, pe)


def _mm_body(x_ref, w_ref, b_ref, o_ref, *, act):
    acc = jnp.dot(x_ref[...], w_ref[...], preferred_element_type=F32)
    acc = acc + b_ref[...]
    if act == "relu":
        acc = jnp.maximum(acc, 0.0)
    elif act == "sigmoid":
        acc = 1.0 / (1.0 + jnp.exp(-acc))
    o_ref[...] = acc


def _matmul(x, w, b, act="none", mb=256):
    M, K = x.shape
    _, N = w.shape
    mb = min(mb, M)
    return pl.pallas_call(
        functools.partial(_mm_body, act=act),
        grid=(M // mb,),
        in_specs=[
            pl.BlockSpec((mb, K), lambda i: (i, 0)),
            pl.BlockSpec((K, N), lambda i: (0, 0)),
            pl.BlockSpec((1, N), lambda i: (0, 0)),
        ],
        out_specs=pl.BlockSpec((mb, N), lambda i: (i, 0)),
        out_shape=jax.ShapeDtypeStruct((M, N), F32),
    )(x, w, b.reshape(1, N))


def _attn_body(q_ref, k_ref, v_ref, o_ref, *, sb, dh, S):
    i = pl.program_id(1)
    q = q_ref[...]
    k = k_ref[...]
    v = v_ref[...]
    s = lax.dot_general(q, k, (((1,), (1,)), ((), ())), preferred_element_type=F32)
    s = s * (1.0 / math.sqrt(dh))
    rows = lax.broadcasted_iota(jnp.int32, (sb, S), 0) + i * sb
    cols = lax.broadcasted_iota(jnp.int32, (sb, S), 1)
    s = jnp.where(cols > rows, -1e30, s)
    m = jnp.max(s, axis=1, keepdims=True)
    p = jnp.exp(s - m)
    p = p / jnp.sum(p, axis=1, keepdims=True)
    o_ref[...] = jnp.dot(p, v, preferred_element_type=F32)


def _attention(qkv, S, D, nhead, sb=256):
    dh = D // nhead
    return pl.pallas_call(
        functools.partial(_attn_body, sb=sb, dh=dh, S=S),
        grid=(nhead, S // sb),
        in_specs=[
            pl.BlockSpec((sb, dh), lambda h, i: (i, h)),
            pl.BlockSpec((S, dh), lambda h, i: (0, nhead + h)),
            pl.BlockSpec((S, dh), lambda h, i: (0, 2 * nhead + h)),
        ],
        out_specs=pl.BlockSpec((sb, dh), lambda h, i: (i, h)),
        out_shape=jax.ShapeDtypeStruct((S, D), F32),
    )(qkv, qkv, qkv)


def _addln_body(a_ref, b_ref, w_ref, bb_ref, o_ref):
    x = a_ref[...] + b_ref[...]
    m = jnp.mean(x, axis=1, keepdims=True)
    var = jnp.mean((x - m) ** 2, axis=1, keepdims=True)
    o_ref[...] = (x - m) / jnp.sqrt(var + 1e-5) * w_ref[...] + bb_ref[...]


def _add_ln(a, b, w, bias, mb=256):
    M, D = a.shape
    return pl.pallas_call(
        _addln_body,
        grid=(M // mb,),
        in_specs=[
            pl.BlockSpec((mb, D), lambda i: (i, 0)),
            pl.BlockSpec((mb, D), lambda i: (i, 0)),
            pl.BlockSpec((1, D), lambda i: (0, 0)),
            pl.BlockSpec((1, D), lambda i: (0, 0)),
        ],
        out_specs=pl.BlockSpec((mb, D), lambda i: (i, 0)),
        out_shape=jax.ShapeDtypeStruct((M, D), F32),
    )(a, b, w.reshape(1, D), bias.reshape(1, D))


def _topk_body(s_ref, o_ref, *, nq):
    s = s_ref[...]
    R, C = s.shape
    flat = (lax.broadcasted_iota(jnp.int32, (R, C), 0) * C
            + lax.broadcasted_iota(jnp.int32, (R, C), 1))
    rowi = lax.broadcasted_iota(jnp.int32, (64, 128), 0)
    coli = lax.broadcasted_iota(jnp.int32, (64, 128), 1)

    def body(t, carry):
        s, o = carry
        m = jnp.max(s)
        cand = jnp.where(s == m, flat, jnp.int32(2 ** 30))
        amin = jnp.min(cand)
        o = o + jnp.where(rowi == t, m, 0.0)
        s = jnp.where(flat == amin, jnp.float32(-1e30), s)
        return s, o

    s, o = lax.fori_loop(0, nq, body, (s, jnp.zeros((64, 128), F32)))
    total = jnp.sum(jnp.where(coli == 0, o, 0.0))
    o_ref[...] = o / total


def _topk(score2d, nq):
    return pl.pallas_call(
        functools.partial(_topk_body, nq=nq),
        out_shape=jax.ShapeDtypeStruct((64, 128), F32),
    )(score2d)


def _contrib_body(w_ref, rv_ref, o_ref):
    o_ref[...] = jnp.log(w_ref[0, 0, 0] * rv_ref[...] + 1e-40)


def _contrib(w2d, rv2, cb=8192):
    NQ_, L2 = rv2.shape
    nc = L2 // cb
    w3 = w2d.reshape(64, 1, 128)
    rv3 = rv2.reshape(NQ_, nc, cb)
    out = pl.pallas_call(
        _contrib_body,
        grid=(NQ_,),
        in_specs=[
            pl.BlockSpec((1, 1, 128), lambda q: (q, 0, 0)),
            pl.BlockSpec((1, nc, cb), lambda q: (q, 0, 0)),
        ],
        out_specs=pl.BlockSpec((1, nc, cb), lambda q: (q, 0, 0)),
        out_shape=jax.ShapeDtypeStruct((NQ_, nc, cb), F32),
    )(w3, rv3)
    return out.reshape(NQ_, L2)


def _loss_body(p_ref, a_ref, o_ref, *, V):
    part = jnp.sum(p_ref[...], axis=1, keepdims=True)
    alab = a_ref[...][:, 0:1]
    lr = jnp.log(V * math.e + part) - 1.0 - alab
    o_ref[...] = jnp.mean(lr).reshape(1, 1)


def _loss(part, alab, V):
    return pl.pallas_call(
        functools.partial(_loss_body, V=V),
        out_shape=jax.ShapeDtypeStruct((1, 1), F32),
    )(part, alab)


def _sc_mesh():
    return plsc.VectorSubcoreMesh(
        core_axis_name="c", subcore_axis_name="s",
        num_cores=_NC, num_subcores=_NS)


def _sc_embed_gather(idx, emb):
    (Sn,) = idx.shape
    V, D = emb.shape
    bpw = Sn // _NW

    @functools.partial(
        pl.kernel, mesh=_sc_mesh(),
        out_type=jax.ShapeDtypeStruct((Sn, D), F32),
        scratch_types=[
            pltpu.VMEM((bpw,), jnp.int32),
            pltpu.VMEM((bpw, D), F32),
            pltpu.SemaphoreType.DMA,
        ],
    )
    def k(idx_hbm, emb_hbm, out_hbm, idx_v, rows_v, sem):
        wid = lax.axis_index("s") * _NC + lax.axis_index("c")
        base = wid * bpw
        pltpu.sync_copy(idx_hbm.at[pl.ds(base, bpw)], idx_v)
        pltpu.async_copy(emb_hbm.at[idx_v], rows_v, sem).wait()
        pltpu.sync_copy(rows_v, out_hbm.at[pl.ds(base, bpw)])

    return k(idx, emb)


def _sc_ce(contrib1, ridx1, labels, V, NQ_, TK):
    (Sn,) = labels.shape
    bpw = Sn // _NW
    Vp = ((V + _NL - 1) // _NL) * _NL
    nzero = Vp // _NL
    E = math.e
    R = 2
    seg = R * TK
    nblk = bpw // R
    stride = Sn * TK
    nck = TK // _NL

    @functools.partial(
        pl.kernel, mesh=_sc_mesh(),
        compiler_params=pltpu.CompilerParams(needs_layout_passes=False),
        out_type=(jax.ShapeDtypeStruct((Sn, _NL), F32),
                  jax.ShapeDtypeStruct((Sn, _NL), F32)),
        scratch_types=[
            pltpu.VMEM((Vp,), F32),
            pltpu.VMEM((Vp,), F32),
            pltpu.VMEM((NQ_ * seg,), F32),
            pltpu.VMEM((NQ_ * seg,), jnp.int32),
            pltpu.VMEM((bpw,), jnp.int32),
            pltpu.VMEM((bpw, _NL), F32),
            pltpu.VMEM((bpw, _NL), F32),
            pltpu.SemaphoreType.DMA,
            pltpu.SemaphoreType.DMA,
        ],
    )
    def k(c_hbm, i_hbm, lab_hbm, part_hbm, alab_hbm,
          accum, cnt, vbuf, ibuf, labv, pout, aout, sem1, sem2):
        wid = lax.axis_index("s") * _NC + lax.axis_index("c")
        base = wid * bpw
        pltpu.sync_copy(lab_hbm.at[pl.ds(base, bpw)], labv)
        zeros16 = jnp.zeros((_NL,), F32)
        ones16 = jnp.ones((_NL,), F32)

        def zbody(t, carry):
            accum[pl.ds(t * _NL, _NL)] = zeros16
            cnt[pl.ds(t * _NL, _NL)] = zeros16
            return carry

        lax.fori_loop(0, nzero, zbody, 0)

        def blk(t, carry):
            s0 = base + t * R

            def fire(q, carry):
                off = q * stride + s0 * TK
                pltpu.async_copy(c_hbm.at[pl.ds(off, seg)],
                                 vbuf.at[pl.ds(q * seg, seg)], sem1)
                pltpu.async_copy(i_hbm.at[pl.ds(off, seg)],
                                 ibuf.at[pl.ds(q * seg, seg)], sem2)
                return carry

            lax.fori_loop(0, NQ_, fire, 0)
            pltpu.make_async_copy(c_hbm.at[pl.ds(0, NQ_ * seg)], vbuf, sem1).wait()
            pltpu.make_async_copy(i_hbm.at[pl.ds(0, NQ_ * seg)], ibuf, sem2).wait()

            for r in range(R):
                i = t * R + r

                def pass_a(q, carry):
                    for kk in range(nck):
                        o = q * seg + r * TK + kk * _NL
                        iv = ibuf[pl.ds(o, _NL)]
                        vv = vbuf[pl.ds(o, _NL)]
                        plsc.addupdate_scatter(accum, [iv], vv)
                        plsc.addupdate_scatter(cnt, [iv], ones16)
                    return carry

                lax.fori_loop(0, NQ_, pass_a, 0)

                ivec = jnp.full((_NL,), i, jnp.int32)
                lab = plsc.load_gather(labv, [ivec])
                aout[i] = plsc.load_gather(accum, [lab])

                def pass_b(q, acc):
                    for kk in range(nck):
                        o = q * seg + r * TK + kk * _NL
                        iv = ibuf[pl.ds(o, _NL)]
                        a = plsc.load_gather(accum, [iv])
                        c = plsc.load_gather(cnt, [iv])
                        acc = acc + (jnp.exp(a + 1.0) - E) / c
                    return acc

                acc = lax.fori_loop(0, NQ_, pass_b, jnp.zeros((_NL,), F32))
                pout[i] = acc

                def pass_c(q, carry):
                    for kk in range(nck):
                        o = q * seg + r * TK + kk * _NL
                        iv = ibuf[pl.ds(o, _NL)]
                        plsc.store_scatter(accum, [iv], zeros16)
                        plsc.store_scatter(cnt, [iv], zeros16)
                    return carry

                lax.fori_loop(0, NQ_, pass_c, 0)
            return carry

        lax.fori_loop(0, nblk, blk, 0)
        pltpu.sync_copy(pout, part_hbm.at[pl.ds(base, bpw)])
        pltpu.sync_copy(aout, alab_hbm.at[pl.ds(base, bpw)])

    return k(contrib1, ridx1, labels)


def kernel(inputs, response_values, response_indices, emb, gates_w, gates_b, layers):
    B_, S_ = inputs.shape
    V_, D_ = emb.shape
    NQ_, _, _, TK = response_values.shape
    nhead = 2
    nhid = layers[0]["ff1_w"].shape[0]
    nhid_p = 256

    idx = inputs.reshape(S_).astype(jnp.int32)
    x0 = _sc_embed_gather(idx, emb)
    x = _scale_pe(x0, _posenc(S_, D_), math.sqrt(D_))

    for p in layers:
        qkv = _matmul(x, p["in_w"].T, p["in_b"])
        attn = _attention(qkv, S_, D_, nhead)
        proj = _matmul(attn, p["out_w"].T, p["out_b"])
        x = _add_ln(x, proj, p["ln1_w"], p["ln1_b"])
        f1w = jnp.zeros((D_, nhid_p), F32).at[:, :nhid].set(p["ff1_w"].T)
        f1b = jnp.zeros((nhid_p,), F32).at[:nhid].set(p["ff1_b"])
        h = _matmul(x, f1w, f1b, act="relu")
        f2w = jnp.zeros((nhid_p, D_), F32).at[:nhid].set(p["ff2_w"].T)
        f = _matmul(h, f2w, p["ff2_b"])
        x = _add_ln(x, f, p["ln2_w"], p["ln2_b"])

    xl = x[S_ - 1:S_, :]
    score = _matmul(xl, gates_w.T, gates_b, act="sigmoid", mb=1)
    routing_score = score.reshape(-1)

    w2d = _topk(score.reshape(8, -1), NQ_)
    rv2 = response_values.reshape(NQ_, S_ * TK)
    ri2 = response_indices.reshape(NQ_, S_ * TK).astype(jnp.int32)
    contrib2 = _contrib(w2d, rv2)
    part, alab = _sc_ce(contrib2.reshape(-1), ri2.reshape(-1), idx, V_, NQ_, TK)
    loss = _loss(part, alab, V_)
    return loss.reshape(()), routing_score


# speedup vs baseline: 27.5226x; 1.1110x over previous
import functools
import math

import numpy as np
import jax
import jax.numpy as jnp
from jax import lax
from jax.experimental import pallas as pl
from jax.experimental.pallas import tpu as pltpu
from jax.experimental.pallas import tpu_sc as plsc

F32 = jnp.float32

_NC, _NS, _NL = 2, 16, 16
_NW = _NC * _NS


def _posenc(seq, dim):
    pos = np.arange(seq)[:, None].astype(np.float32)
    div = np.exp(np.arange(0, dim, 2).astype(np.float32) * (-math.log(10000.0) / dim))
    pe = np.zeros((seq, dim), np.float32)
    pe[:, 0::2] = np.sin(pos * div)
    pe[:, 1::2] = np.cos(pos * div)
    return jnp.asarray(pe)


def _scalepe_body(x_ref, p_ref, o_ref, *, scale):
    o_ref[...] = x_ref[...] * scale + p_ref[...]


def _scale_pe(x, pe, scale, mb=256):
    M, D = x.shape
    return pl.pallas_call(
        functools.partial(_scalepe_body, scale=scale),
        grid=(M // mb,),
        in_specs=[
            pl.BlockSpec((mb, D), lambda i: (i, 0)),
            pl.BlockSpec((mb, D), lambda i: (i, 0)),
        ],
        out_specs=pl.BlockSpec((mb, D), lambda i: (i, 0)),
        out_shape=jax.ShapeDtypeStruct((M, D), F32),
    )(x, pe)


def _mm_body(x_ref, w_ref, b_ref, o_ref, *, act):
    acc = jnp.dot(x_ref[...], w_ref[...], preferred_element_type=F32)
    acc = acc + b_ref[...]
    if act == "relu":
        acc = jnp.maximum(acc, 0.0)
    elif act == "sigmoid":
        acc = 1.0 / (1.0 + jnp.exp(-acc))
    o_ref[...] = acc


def _matmul(x, w, b, act="none", mb=256):
    M, K = x.shape
    _, N = w.shape
    mb = min(mb, M)
    return pl.pallas_call(
        functools.partial(_mm_body, act=act),
        grid=(M // mb,),
        in_specs=[
            pl.BlockSpec((mb, K), lambda i: (i, 0)),
            pl.BlockSpec((K, N), lambda i: (0, 0)),
            pl.BlockSpec((1, N), lambda i: (0, 0)),
        ],
        out_specs=pl.BlockSpec((mb, N), lambda i: (i, 0)),
        out_shape=jax.ShapeDtypeStruct((M, N), F32),
    )(x, w, b.reshape(1, N))


def _attn_body(q_ref, k_ref, v_ref, o_ref, *, sb, dh, S):
    i = pl.program_id(1)
    q = q_ref[...]
    k = k_ref[...]
    v = v_ref[...]
    s = lax.dot_general(q, k, (((1,), (1,)), ((), ())), preferred_element_type=F32)
    s = s * (1.0 / math.sqrt(dh))
    rows = lax.broadcasted_iota(jnp.int32, (sb, S), 0) + i * sb
    cols = lax.broadcasted_iota(jnp.int32, (sb, S), 1)
    s = jnp.where(cols > rows, -1e30, s)
    m = jnp.max(s, axis=1, keepdims=True)
    p = jnp.exp(s - m)
    p = p / jnp.sum(p, axis=1, keepdims=True)
    o_ref[...] = jnp.dot(p, v, preferred_element_type=F32)


def _attention(qkv, S, D, nhead, sb=256):
    dh = D // nhead
    return pl.pallas_call(
        functools.partial(_attn_body, sb=sb, dh=dh, S=S),
        grid=(nhead, S // sb),
        in_specs=[
            pl.BlockSpec((sb, dh), lambda h, i: (i, h)),
            pl.BlockSpec((S, dh), lambda h, i: (0, nhead + h)),
            pl.BlockSpec((S, dh), lambda h, i: (0, 2 * nhead + h)),
        ],
        out_specs=pl.BlockSpec((sb, dh), lambda h, i: (i, h)),
        out_shape=jax.ShapeDtypeStruct((S, D), F32),
    )(qkv, qkv, qkv)


def _addln_body(a_ref, b_ref, w_ref, bb_ref, o_ref):
    x = a_ref[...] + b_ref[...]
    m = jnp.mean(x, axis=1, keepdims=True)
    var = jnp.mean((x - m) ** 2, axis=1, keepdims=True)
    o_ref[...] = (x - m) / jnp.sqrt(var + 1e-5) * w_ref[...] + bb_ref[...]


def _add_ln(a, b, w, bias, mb=256):
    M, D = a.shape
    return pl.pallas_call(
        _addln_body,
        grid=(M // mb,),
        in_specs=[
            pl.BlockSpec((mb, D), lambda i: (i, 0)),
            pl.BlockSpec((mb, D), lambda i: (i, 0)),
            pl.BlockSpec((1, D), lambda i: (0, 0)),
            pl.BlockSpec((1, D), lambda i: (0, 0)),
        ],
        out_specs=pl.BlockSpec((mb, D), lambda i: (i, 0)),
        out_shape=jax.ShapeDtypeStruct((M, D), F32),
    )(a, b, w.reshape(1, D), bias.reshape(1, D))


def _topk_body(s_ref, o_ref, *, nq):
    s = s_ref[...]
    R, C = s.shape
    flat = (lax.broadcasted_iota(jnp.int32, (R, C), 0) * C
            + lax.broadcasted_iota(jnp.int32, (R, C), 1))
    rowi = lax.broadcasted_iota(jnp.int32, (64, 128), 0)
    coli = lax.broadcasted_iota(jnp.int32, (64, 128), 1)

    def body(t, carry):
        s, o = carry
        m = jnp.max(s)
        cand = jnp.where(s == m, flat, jnp.int32(2 ** 30))
        amin = jnp.min(cand)
        o = o + jnp.where(rowi == t, m, 0.0)
        s = jnp.where(flat == amin, jnp.float32(-1e30), s)
        return s, o

    s, o = lax.fori_loop(0, nq, body, (s, jnp.zeros((64, 128), F32)))
    total = jnp.sum(jnp.where(coli == 0, o, 0.0))
    o_ref[...] = o / total


def _topk(score2d, nq):
    return pl.pallas_call(
        functools.partial(_topk_body, nq=nq),
        out_shape=jax.ShapeDtypeStruct((64, 128), F32),
    )(score2d)


def _contrib_body(w_ref, rv_ref, o_ref):
    o_ref[...] = jnp.log(w_ref[0, 0, 0] * rv_ref[...] + 1e-40)


def _contrib(w2d, rv2, cb=8192):
    NQ_, L2 = rv2.shape
    nc = L2 // cb
    w3 = w2d.reshape(64, 1, 128)
    rv3 = rv2.reshape(NQ_, nc, cb)
    out = pl.pallas_call(
        _contrib_body,
        grid=(NQ_,),
        in_specs=[
            pl.BlockSpec((1, 1, 128), lambda q: (q, 0, 0)),
            pl.BlockSpec((1, nc, cb), lambda q: (q, 0, 0)),
        ],
        out_specs=pl.BlockSpec((1, nc, cb), lambda q: (q, 0, 0)),
        out_shape=jax.ShapeDtypeStruct((NQ_, nc, cb), F32),
    )(w3, rv3)
    return out.reshape(NQ_, L2)


def _loss_body(p_ref, a_ref, o_ref, *, V):
    part = jnp.sum(p_ref[...], axis=1, keepdims=True)
    alab = a_ref[...][:, 0:1]
    lr = jnp.log(V * math.e + part) - 1.0 - alab
    o_ref[...] = jnp.mean(lr).reshape(1, 1)


def _loss(part, alab, V):
    return pl.pallas_call(
        functools.partial(_loss_body, V=V),
        out_shape=jax.ShapeDtypeStruct((1, 1), F32),
    )(part, alab)


def _sc_mesh():
    return plsc.VectorSubcoreMesh(
        core_axis_name="c", subcore_axis_name="s",
        num_cores=_NC, num_subcores=_NS)


def _sc_embed_gather(idx, emb):
    (Sn,) = idx.shape
    V, D = emb.shape
    bpw = Sn // _NW

    @functools.partial(
        pl.kernel, mesh=_sc_mesh(),
        out_type=jax.ShapeDtypeStruct((Sn, D), F32),
        scratch_types=[
            pltpu.VMEM((bpw,), jnp.int32),
            pltpu.VMEM((bpw, D), F32),
            pltpu.SemaphoreType.DMA,
        ],
    )
    def k(idx_hbm, emb_hbm, out_hbm, idx_v, rows_v, sem):
        wid = lax.axis_index("s") * _NC + lax.axis_index("c")
        base = wid * bpw
        pltpu.sync_copy(idx_hbm.at[pl.ds(base, bpw)], idx_v)
        pltpu.async_copy(emb_hbm.at[idx_v], rows_v, sem).wait()
        pltpu.sync_copy(rows_v, out_hbm.at[pl.ds(base, bpw)])

    return k(idx, emb)


def _sc_ce(contrib1, ridx1, labels, V, NQ_, TK):
    (Sn,) = labels.shape
    bpw = Sn // _NW
    Vp = ((V + _NL - 1) // _NL) * _NL
    nzero = Vp // _NL
    E = math.e
    R = 4
    seg = R * TK
    nblk = bpw // R
    stride = Sn * TK
    nck = TK // _NL

    @functools.partial(
        pl.kernel, mesh=_sc_mesh(),
        compiler_params=pltpu.CompilerParams(needs_layout_passes=False),
        out_type=(jax.ShapeDtypeStruct((Sn, _NL), F32),
                  jax.ShapeDtypeStruct((Sn, _NL), F32)),
        scratch_types=[
            pltpu.VMEM((Vp,), F32),
            pltpu.VMEM((NQ_ * seg,), F32),
            pltpu.VMEM((NQ_ * seg,), jnp.int32),
            pltpu.VMEM((NQ_ * seg,), F32),
            pltpu.VMEM((NQ_ * seg,), jnp.int32),
            pltpu.VMEM((bpw,), jnp.int32),
            pltpu.VMEM((bpw, _NL), F32),
            pltpu.VMEM((bpw, _NL), F32),
            pltpu.SemaphoreType.DMA,
            pltpu.SemaphoreType.DMA,
            pltpu.SemaphoreType.DMA,
            pltpu.SemaphoreType.DMA,
        ],
    )
    def k(c_hbm, i_hbm, lab_hbm, part_hbm, alab_hbm,
          accum, vbufa, ibufa, vbufb, ibufb, labv, pout, aout,
          sva, sia, svb, sib):
        wid = lax.axis_index("s") * _NC + lax.axis_index("c")
        base = wid * bpw
        pltpu.sync_copy(lab_hbm.at[pl.ds(base, bpw)], labv)
        zeros16 = jnp.zeros((_NL,), F32)

        def zbody(t, carry):
            accum[pl.ds(t * _NL, _NL)] = zeros16
            return carry

        lax.fori_loop(0, nzero, zbody, 0)

        def fire(t, vbuf, ibuf, sv, si):
            s0 = base + t * R

            def fq(q, carry):
                off = q * stride + s0 * TK
                pltpu.async_copy(c_hbm.at[pl.ds(off, seg)],
                                 vbuf.at[pl.ds(q * seg, seg)], sv)
                pltpu.async_copy(i_hbm.at[pl.ds(off, seg)],
                                 ibuf.at[pl.ds(q * seg, seg)], si)
                return carry

            lax.fori_loop(0, NQ_, fq, 0)

        def process(t, vbuf, ibuf, sv, si):
            pltpu.make_async_copy(c_hbm.at[pl.ds(0, NQ_ * seg)], vbuf, sv).wait()
            pltpu.make_async_copy(i_hbm.at[pl.ds(0, NQ_ * seg)], ibuf, si).wait()
            for r in range(R):
                i = t * R + r

                def pass_a(q, carry):
                    for kk in range(nck):
                        o = q * seg + r * TK + kk * _NL
                        iv = ibuf[pl.ds(o, _NL)]
                        vv = vbuf[pl.ds(o, _NL)]
                        plsc.addupdate_scatter(accum, [iv], vv)
                    return carry

                lax.fori_loop(0, NQ_, pass_a, 0)

                ivec = jnp.full((_NL,), i, jnp.int32)
                lab = plsc.load_gather(labv, [ivec])
                aout[i] = plsc.load_gather(accum, [lab])

                def pass_b(q, acc):
                    for kk in range(nck):
                        o = q * seg + r * TK + kk * _NL
                        iv = ibuf[pl.ds(o, _NL)]
                        a = plsc.load_gather(accum, [iv])
                        _, lastm = plsc.scan_count(iv)
                        term = jnp.exp(a + 1.0) - E
                        acc = acc + jnp.where(lastm, term, 0.0)
                        plsc.store_scatter(accum, [iv], zeros16)
                    return acc

                acc = lax.fori_loop(0, NQ_, pass_b, jnp.zeros((_NL,), F32))
                pout[i] = acc

        fire(0, vbufa, ibufa, sva, sia)

        def blk2(u, carry):
            t0 = 2 * u
            fire(t0 + 1, vbufb, ibufb, svb, sib)
            process(t0, vbufa, ibufa, sva, sia)

            @pl.when(t0 + 2 < nblk)
            def _():
                fire(t0 + 2, vbufa, ibufa, sva, sia)

            process(t0 + 1, vbufb, ibufb, svb, sib)
            return carry

        lax.fori_loop(0, nblk // 2, blk2, 0)
        pltpu.sync_copy(pout, part_hbm.at[pl.ds(base, bpw)])
        pltpu.sync_copy(aout, alab_hbm.at[pl.ds(base, bpw)])

    return k(contrib1, ridx1, labels)


def kernel(inputs, response_values, response_indices, emb, gates_w, gates_b, layers):
    B_, S_ = inputs.shape
    V_, D_ = emb.shape
    NQ_, _, _, TK = response_values.shape
    nhead = 2
    nhid = layers[0]["ff1_w"].shape[0]
    nhid_p = 256

    idx = inputs.reshape(S_).astype(jnp.int32)
    x0 = _sc_embed_gather(idx, emb)
    x = _scale_pe(x0, _posenc(S_, D_), math.sqrt(D_))

    for p in layers:
        qkv = _matmul(x, p["in_w"].T, p["in_b"])
        attn = _attention(qkv, S_, D_, nhead)
        proj = _matmul(attn, p["out_w"].T, p["out_b"])
        x = _add_ln(x, proj, p["ln1_w"], p["ln1_b"])
        f1w = jnp.zeros((D_, nhid_p), F32).at[:, :nhid].set(p["ff1_w"].T)
        f1b = jnp.zeros((nhid_p,), F32).at[:nhid].set(p["ff1_b"])
        h = _matmul(x, f1w, f1b, act="relu")
        f2w = jnp.zeros((nhid_p, D_), F32).at[:nhid].set(p["ff2_w"].T)
        f = _matmul(h, f2w, p["ff2_b"])
        x = _add_ln(x, f, p["ln2_w"], p["ln2_b"])

    xl = x[S_ - 1:S_, :]
    score = _matmul(xl, gates_w.T, gates_b, act="sigmoid", mb=1)
    routing_score = score.reshape(-1)

    w2d = _topk(score.reshape(8, -1), NQ_)
    rv2 = response_values.reshape(NQ_, S_ * TK)
    ri2 = response_indices.reshape(NQ_, S_ * TK).astype(jnp.int32)
    contrib2 = _contrib(w2d, rv2)
    part, alab = _sc_ce(contrib2.reshape(-1), ri2.reshape(-1), idx, V_, NQ_, TK)
    loss = _loss(part, alab, V_)
    return loss.reshape(()), routing_score
